# initial kernel scaffold (unmeasured)
import jax
import jax.numpy as jnp
from jax import lax
from jax.experimental import pallas as pl
from jax.experimental.pallas import tpu as pltpu

B = 8
H = 8
D = 128
PAGE = 16
NPAGES = 512
NSLOTS = 512
ZDIM = 4
BP = 32
NB = NPAGES // BP
KTOK = BP * PAGE
NEG = -1e30
SCALE = D ** -0.5


def kernel(Q, K, V, bt, lens):
    lens2 = lens.reshape(B, 1)

    def body(q_ref, k_ref, v_ref, bt_ref, lens_ref, o_ref,
             m_scr, l_scr, acc_scr, acc_comm, ml_comm,
             acc_send, acc_recv, ml_send, ml_recv):
        pb = pl.program_id(0)
        my_x = lax.axis_index("x")
        my_y = lax.axis_index("y")
        my_z = lax.axis_index("z")

        @pl.when(pb == 0)
        def _init():
            m_scr[...] = jnp.full((H, B), NEG, jnp.float32)
            l_scr[...] = jnp.zeros((H, B), jnp.float32)
            acc_scr[...] = jnp.zeros((H, B, D), jnp.float32)

        bt_ = bt_ref[...]
        lens_ = lens_ref[...]
        base = my_z * NPAGES + pb * BP
        gp = base + lax.broadcasted_iota(jnp.int32, (1, BP, 1), 1)
        slot = lax.broadcasted_iota(jnp.int32, (1, 1, NSLOTS), 2)
        sel = (bt_[:, None, :] == gp) & (slot < lens_[:, :, None])
        count = sel.astype(jnp.int32).sum(axis=2).astype(jnp.float32)
        count_tok = jnp.broadcast_to(
            count[:, :, None], (B, BP, PAGE)).reshape(B, KTOK)

        q = (q_ref[...].reshape(B, H, D) * SCALE).astype(jnp.bfloat16)
        kb = k_ref[...].reshape(KTOK, H, D).astype(jnp.bfloat16)
        s = lax.dot_general(q, kb, (((2,), (2,)), ((1,), (1,))),
                            preferred_element_type=jnp.float32)
        sm = jnp.where((count_tok > 0.0)[None], s, NEG)
        m_old = m_scr[...]
        m_new = jnp.maximum(m_old, jnp.max(sm, axis=2))
        alpha = jnp.exp(m_old - m_new)
        p = count_tok[None] * jnp.exp(sm - m_new[:, :, None])
        l_scr[...] = l_scr[...] * alpha + jnp.sum(p, axis=2)
        vb = v_ref[...].reshape(KTOK, H, D).astype(jnp.bfloat16)
        pv = lax.dot_general(p.astype(jnp.bfloat16), vb,
                             (((2,), (0,)), ((0,), (1,))),
                             preferred_element_type=jnp.float32)
        acc_scr[...] = acc_scr[...] * alpha[:, :, None] + pv
        m_scr[...] = m_new

        @pl.when(pb == NB - 1)
        def _exchange():
            acc_comm[pl.ds(my_z, 1)] = acc_scr[...][None]
            ml_comm[pl.ds(my_z, 1)] = jnp.stack([m_scr[...], l_scr[...]])[None]

            sends = []
            for dz in range(1, ZDIM):
                zt = (my_z + dz) % ZDIM
                for comm, ssem, rsem in ((acc_comm, acc_send, acc_recv),
                                         (ml_comm, ml_send, ml_recv)):
                    r = pltpu.make_async_remote_copy(
                        src_ref=comm.at[my_z],
                        dst_ref=comm.at[my_z],
                        send_sem=ssem.at[zt],
                        recv_sem=rsem.at[my_z],
                        device_id=(my_x, my_y, zt),
                        device_id_type=pl.DeviceIdType.MESH,
                    )
                    r.start()
                    sends.append(r)

            for dz in range(1, ZDIM):
                zs = (my_z + dz) % ZDIM
                for comm, ssem, rsem in ((acc_comm, acc_send, acc_recv),
                                         (ml_comm, ml_send, ml_recv)):
                    r = pltpu.make_async_remote_copy(
                        src_ref=comm.at[zs],
                        dst_ref=comm.at[zs],
                        send_sem=ssem.at[zs],
                        recv_sem=rsem.at[zs],
                        device_id=(my_x, my_y, zs),
                        device_id_type=pl.DeviceIdType.MESH,
                    )
                    r.wait_recv()
            for r in sends:
                r.wait_send()

            ml = ml_comm[...]
            m_z = ml[:, 0]
            l_z = ml[:, 1]
            mg = jnp.max(m_z, axis=0)
            sc = jnp.exp(m_z - mg[None])
            lg = jnp.sum(l_z * sc, axis=0)
            og = jnp.sum(acc_comm[...] * sc[:, :, :, None], axis=0)
            out = og / lg[:, :, None]
            o_ref[...] = out.transpose(1, 0, 2)[:, None]

    return pl.pallas_call(
        body,
        grid=(NB,),
        in_specs=[
            pl.BlockSpec((B, 1, H, D), lambda i: (0, 0, 0, 0)),
            pl.BlockSpec((BP, PAGE, H, D), lambda i: (i, 0, 0, 0)),
            pl.BlockSpec((BP, PAGE, H, D), lambda i: (i, 0, 0, 0)),
            pl.BlockSpec((B, NSLOTS), lambda i: (0, 0)),
            pl.BlockSpec((B, 1), lambda i: (0, 0)),
        ],
        out_specs=pl.BlockSpec((B, 1, H, D), lambda i: (0, 0, 0, 0)),
        out_shape=jax.ShapeDtypeStruct((B, 1, H, D), jnp.float32),
        scratch_shapes=[
            pltpu.VMEM((H, B), jnp.float32),
            pltpu.VMEM((H, B), jnp.float32),
            pltpu.VMEM((H, B, D), jnp.float32),
            pltpu.VMEM((ZDIM, H, B, D), jnp.float32),
            pltpu.VMEM((ZDIM, 2, H, B), jnp.float32),
            pltpu.SemaphoreType.DMA((ZDIM,)),
            pltpu.SemaphoreType.DMA((ZDIM,)),
            pltpu.SemaphoreType.DMA((ZDIM,)),
            pltpu.SemaphoreType.DMA((ZDIM,)),
        ],
        compiler_params=pltpu.CompilerParams(
            dimension_semantics=("arbitrary",),
            collective_id=0,
        ),
    )(Q, K, V, bt, lens2)


# baseline (device time: 184334 ns/iter reference)
import jax
import jax.numpy as jnp
from jax import lax
from jax.experimental import pallas as pl
from jax.experimental.pallas import tpu as pltpu

B = 8
H = 8
D = 128
PAGE = 16
NPAGES = 512
NSLOTS = 512
ZDIM = 4
BP = 32
NB = NPAGES // BP
KTOK = BP * PAGE
NEG = -1e30
SCALE = D ** -0.5


def kernel(Q, K, V, bt, lens):
    lens2 = lens.reshape(B, 1)

    def body(q_ref, k_ref, v_ref, bt_ref, lens_ref, o_ref,
             m_scr, l_scr, acc_scr, acc_comm, ml_comm,
             acc_send, acc_recv, ml_send, ml_recv):
        pb = pl.program_id(0)
        my_x = lax.axis_index("x")
        my_y = lax.axis_index("y")
        my_z = lax.axis_index("z")

        @pl.when(pb == 0)
        def _init():
            m_scr[...] = jnp.full((H, B), NEG, jnp.float32)
            l_scr[...] = jnp.zeros((H, B), jnp.float32)
            acc_scr[...] = jnp.zeros((H, B, D), jnp.float32)

        bt_ = bt_ref[...]
        lens_ = lens_ref[...]
        base = my_z * NPAGES + pb * BP
        gp = base + lax.broadcasted_iota(jnp.int32, (1, BP, 1), 1)
        slot = lax.broadcasted_iota(jnp.int32, (1, 1, NSLOTS), 2)
        sel = (bt_[:, None, :] == gp) & (slot < lens_[:, :, None])
        count = sel.astype(jnp.int32).sum(axis=2).astype(jnp.float32)
        count_tok = jnp.broadcast_to(
            count[:, :, None], (B, BP, PAGE)).reshape(B, KTOK)

        q = (q_ref[...].reshape(B, H, D) * SCALE).astype(jnp.bfloat16)
        kb = k_ref[...].reshape(KTOK, H, D).astype(jnp.bfloat16)
        s = lax.dot_general(q, kb, (((2,), (2,)), ((1,), (1,))),
                            preferred_element_type=jnp.float32)
        sm = jnp.where((count_tok > 0.0)[None], s, NEG)
        m_old = m_scr[...]
        m_new = jnp.maximum(m_old, jnp.max(sm, axis=2))
        alpha = jnp.exp(m_old - m_new)
        p = count_tok[None] * jnp.exp(sm - m_new[:, :, None])
        l_scr[...] = l_scr[...] * alpha + jnp.sum(p, axis=2)
        vb = v_ref[...].reshape(KTOK, H, D).astype(jnp.bfloat16)
        pv = lax.dot_general(p.astype(jnp.bfloat16), vb,
                             (((2,), (0,)), ((0,), (1,))),
                             preferred_element_type=jnp.float32)
        acc_scr[...] = acc_scr[...] * alpha[:, :, None] + pv
        m_scr[...] = m_new

        @pl.when(pb == NB - 1)
        def _exchange():
            acc_comm[pl.ds(my_z, 1)] = acc_scr[...][None]
            ml_comm[pl.ds(my_z, 1)] = jnp.stack([m_scr[...], l_scr[...]])[None]

            sends = []
            for dz in range(1, ZDIM):
                zt = (my_z + dz) % ZDIM
                for comm, ssem, rsem in ((acc_comm, acc_send, acc_recv),
                                         (ml_comm, ml_send, ml_recv)):
                    r = pltpu.make_async_remote_copy(
                        src_ref=comm.at[my_z],
                        dst_ref=comm.at[my_z],
                        send_sem=ssem.at[zt],
                        recv_sem=rsem.at[my_z],
                        device_id=(my_x, my_y, zt),
                        device_id_type=pl.DeviceIdType.MESH,
                    )
                    r.start()
                    sends.append(r)

            for dz in range(1, ZDIM):
                zs = (my_z + dz) % ZDIM
                for comm, ssem, rsem in ((acc_comm, acc_send, acc_recv),
                                         (ml_comm, ml_send, ml_recv)):
                    r = pltpu.make_async_remote_copy(
                        src_ref=comm.at[zs],
                        dst_ref=comm.at[zs],
                        send_sem=ssem.at[zs],
                        recv_sem=rsem.at[zs],
                        device_id=(my_x, my_y, zs),
                        device_id_type=pl.DeviceIdType.MESH,
                    )
                    r.wait_recv()
            for r in sends:
                r.wait_send()

            ml = ml_comm[...]
            m_z = ml[:, 0]
            l_z = ml[:, 1]
            mg = jnp.max(m_z, axis=0)
            sc = jnp.exp(m_z - mg[None])
            lg = jnp.sum(l_z * sc, axis=0)
            og = jnp.sum(acc_comm[...] * sc[:, :, :, None], axis=0)
            out = og / lg[:, :, None]
            o_ref[...] = out.transpose(1, 0, 2)[:, None]

    return pl.pallas_call(
        body,
        grid=(NB,),
        in_specs=[
            pl.BlockSpec((B, 1, H, D), lambda i: (0, 0, 0, 0)),
            pl.BlockSpec((BP, PAGE, H, D), lambda i: (i, 0, 0, 0)),
            pl.BlockSpec((BP, PAGE, H, D), lambda i: (i, 0, 0, 0)),
            pl.BlockSpec((B, NSLOTS), lambda i: (0, 0)),
            pl.BlockSpec((B, 1), lambda i: (0, 0)),
        ],
        out_specs=pl.BlockSpec((B, 1, H, D), lambda i: (0, 0, 0, 0)),
        out_shape=jax.ShapeDtypeStruct((B, 1, H, D), jnp.float32),
        scratch_shapes=[
            pltpu.VMEM((H, B), jnp.float32),
            pltpu.VMEM((H, B), jnp.float32),
            pltpu.VMEM((H, B, D), jnp.float32),
            pltpu.VMEM((ZDIM, H, B, D), jnp.float32),
            pltpu.VMEM((ZDIM, 2, H, B), jnp.float32),
            pltpu.SemaphoreType.DMA((ZDIM,)),
            pltpu.SemaphoreType.DMA((ZDIM,)),
            pltpu.SemaphoreType.DMA((ZDIM,)),
            pltpu.SemaphoreType.DMA((ZDIM,)),
        ],
        compiler_params=pltpu.CompilerParams(
            dimension_semantics=("arbitrary",),
        ),
    )(Q, K, V, bt, lens2)


# device time: 148940 ns/iter; 1.2376x vs baseline; 1.2376x over previous
import jax
import jax.numpy as jnp
from jax import lax
from jax.experimental import pallas as pl
from jax.experimental.pallas import tpu as pltpu

B = 8
H = 8
D = 128
PAGE = 16
NPAGES = 512
NSLOTS = 512
ZDIM = 4
BP = 64
NB = NPAGES // BP
KTOK = BP * PAGE
NEG = -1e30
SCALE = D ** -0.5


def kernel(Q, K, V, bt, lens):
    lens2 = lens.reshape(B, 1)
    Q2 = Q.reshape(B, H * D)
    K2 = K.reshape(NPAGES, PAGE, H * D)
    V2 = V.reshape(NPAGES, PAGE, H * D)

    def body(q_ref, k_ref, v_ref, bt_ref, lens_ref, o_ref,
             ct_scr, m_scr, l_scr, acc_scr, acc_comm, ml_comm,
             acc_send, acc_recv, ml_send, ml_recv):
        pb = pl.program_id(0)
        h = pl.program_id(1)
        my_x = lax.axis_index("x")
        my_y = lax.axis_index("y")
        my_z = lax.axis_index("z")

        @pl.when(pb == 0)
        def _init_head():
            m_scr[pl.ds(h, 1)] = jnp.full((1, B, 1), NEG, jnp.float32)
            l_scr[pl.ds(h, 1)] = jnp.zeros((1, B, 1), jnp.float32)
            acc_scr[pl.ds(h, 1)] = jnp.zeros((1, B, D), jnp.float32)

        @pl.when(h == 0)
        def _count():
            bt_ = bt_ref[...]
            lens_ = lens_ref[...]
            base = my_z * NPAGES + pb * BP
            gp = base + lax.broadcasted_iota(jnp.int32, (1, BP, 1), 1)
            slot = lax.broadcasted_iota(jnp.int32, (1, 1, NSLOTS), 2)
            sel = (bt_[:, None, :] == gp) & (slot < lens_[:, :, None])
            count = sel.astype(jnp.int32).sum(axis=2).astype(jnp.float32)
            ct_scr[...] = jnp.broadcast_to(
                count[:, :, None], (B, BP, PAGE)).reshape(B, KTOK)

        q = (q_ref[...] * SCALE).astype(jnp.bfloat16)
        kb = k_ref[...].reshape(KTOK, D).astype(jnp.bfloat16)
        s = lax.dot_general(q, kb, (((1,), (1,)), ((), ())),
                            preferred_element_type=jnp.float32)
        ct = ct_scr[...]
        sm = jnp.where(ct > 0.0, s, NEG)
        m_old = m_scr[pl.ds(h, 1)].reshape(B, 1)
        m_new = jnp.maximum(m_old, jnp.max(sm, axis=1, keepdims=True))
        alpha = jnp.exp(m_old - m_new)
        p = ct * jnp.exp(sm - m_new)
        l_old = l_scr[pl.ds(h, 1)].reshape(B, 1)
        l_scr[pl.ds(h, 1)] = (l_old * alpha
                              + jnp.sum(p, axis=1, keepdims=True))[None]
        vb = v_ref[...].reshape(KTOK, D).astype(jnp.bfloat16)
        pv = lax.dot_general(p.astype(jnp.bfloat16), vb,
                             (((1,), (0,)), ((), ())),
                             preferred_element_type=jnp.float32)
        acc_old = acc_scr[pl.ds(h, 1)].reshape(B, D)
        acc_scr[pl.ds(h, 1)] = (acc_old * alpha + pv)[None]
        m_scr[pl.ds(h, 1)] = m_new[None]

        @pl.when((pb == NB - 1) & (h == H - 1))
        def _exchange():
            acc_comm[pl.ds(my_z, 1)] = acc_scr[...][None]
            ml_comm[pl.ds(my_z, 1)] = jnp.stack(
                [m_scr[...].reshape(H, B), l_scr[...].reshape(H, B)])[None]

            sends = []
            for dz in range(1, ZDIM):
                zt = (my_z + dz) % ZDIM
                for comm, ssem, rsem in ((acc_comm, acc_send, acc_recv),
                                         (ml_comm, ml_send, ml_recv)):
                    r = pltpu.make_async_remote_copy(
                        src_ref=comm.at[my_z],
                        dst_ref=comm.at[my_z],
                        send_sem=ssem.at[zt],
                        recv_sem=rsem.at[my_z],
                        device_id=(my_x, my_y, zt),
                        device_id_type=pl.DeviceIdType.MESH,
                    )
                    r.start()
                    sends.append(r)

            for dz in range(1, ZDIM):
                zs = (my_z + dz) % ZDIM
                for comm, ssem, rsem in ((acc_comm, acc_send, acc_recv),
                                         (ml_comm, ml_send, ml_recv)):
                    r = pltpu.make_async_remote_copy(
                        src_ref=comm.at[zs],
                        dst_ref=comm.at[zs],
                        send_sem=ssem.at[zs],
                        recv_sem=rsem.at[zs],
                        device_id=(my_x, my_y, zs),
                        device_id_type=pl.DeviceIdType.MESH,
                    )
                    r.wait_recv()
            for r in sends:
                r.wait_send()

            ml = ml_comm[...]
            m_z = ml[:, 0]
            l_z = ml[:, 1]
            mg = jnp.max(m_z, axis=0)
            sc = jnp.exp(m_z - mg[None])
            lg = jnp.sum(l_z * sc, axis=0)
            og = jnp.sum(acc_comm[...] * sc[:, :, :, None], axis=0)
            out = og / lg[:, :, None]
            o_ref[...] = out.transpose(1, 0, 2)[:, None]

    return pl.pallas_call(
        body,
        grid=(NB, H),
        in_specs=[
            pl.BlockSpec((B, D), lambda i, h: (0, h)),
            pl.BlockSpec((BP, PAGE, D), lambda i, h: (i, 0, h)),
            pl.BlockSpec((BP, PAGE, D), lambda i, h: (i, 0, h)),
            pl.BlockSpec((B, NSLOTS), lambda i, h: (0, 0)),
            pl.BlockSpec((B, 1), lambda i, h: (0, 0)),
        ],
        out_specs=pl.BlockSpec((B, 1, H, D), lambda i, h: (0, 0, 0, 0)),
        out_shape=jax.ShapeDtypeStruct((B, 1, H, D), jnp.float32),
        scratch_shapes=[
            pltpu.VMEM((B, KTOK), jnp.float32),
            pltpu.VMEM((H, B, 1), jnp.float32),
            pltpu.VMEM((H, B, 1), jnp.float32),
            pltpu.VMEM((H, B, D), jnp.float32),
            pltpu.VMEM((ZDIM, H, B, D), jnp.float32),
            pltpu.VMEM((ZDIM, 2, H, B), jnp.float32),
            pltpu.SemaphoreType.DMA((ZDIM,)),
            pltpu.SemaphoreType.DMA((ZDIM,)),
            pltpu.SemaphoreType.DMA((ZDIM,)),
            pltpu.SemaphoreType.DMA((ZDIM,)),
        ],
        compiler_params=pltpu.CompilerParams(
            dimension_semantics=("arbitrary", "arbitrary"),
        ),
    )(Q2, K2, V2, bt, lens2)


# device time: 118445 ns/iter; 1.5563x vs baseline; 1.2575x over previous
import jax
import jax.numpy as jnp
from jax import lax
from jax.experimental import pallas as pl
from jax.experimental.pallas import tpu as pltpu

B = 8
H = 8
D = 128
PAGE = 16
NPAGES = 512
NSLOTS = 512
ZDIM = 4
BP = 64
NB = NPAGES // BP
KTOK = BP * PAGE
NEG = -1e30
SCALE = D ** -0.5


def kernel(Q, K, V, bt, lens):
    lens2 = lens.reshape(B, 1)
    Q2 = Q.reshape(B, H * D)
    K2 = K.reshape(NPAGES, PAGE, H * D)
    V2 = V.reshape(NPAGES, PAGE, H * D)

    def body(q_ref, k_ref, v_ref, bt_ref, lens_ref, o_ref,
             m_scr, l_scr, acc_scr, acc_comm, ml_comm,
             acc_send, acc_recv, ml_send, ml_recv):
        pb = pl.program_id(0)
        my_x = lax.axis_index("x")
        my_y = lax.axis_index("y")
        my_z = lax.axis_index("z")

        @pl.when(pb == 0)
        def _init():
            m_scr[...] = jnp.full((H, B, 1), NEG, jnp.float32)
            l_scr[...] = jnp.zeros((H, B, 1), jnp.float32)
            acc_scr[...] = jnp.zeros((H, B, D), jnp.float32)

        bt_ = bt_ref[...]
        lens_ = lens_ref[...]
        base = my_z * NPAGES + pb * BP
        gp = base + lax.broadcasted_iota(jnp.int32, (1, BP, 1), 1)
        slot = lax.broadcasted_iota(jnp.int32, (1, 1, NSLOTS), 2)
        sel = (bt_[:, None, :] == gp) & (slot < lens_[:, :, None])
        count = sel.astype(jnp.int32).sum(axis=2).astype(jnp.float32)
        ct = jnp.broadcast_to(
            count[:, :, None], (B, BP, PAGE)).reshape(B, KTOK)
        ct_pos = ct > 0.0

        qf = (q_ref[...] * SCALE).astype(jnp.bfloat16)
        kbf = k_ref[...].reshape(KTOK, H * D).astype(jnp.bfloat16)
        vbf = v_ref[...].reshape(KTOK, H * D).astype(jnp.bfloat16)

        for h in range(H):
            lo, hi = h * D, (h + 1) * D
            s = lax.dot_general(qf[:, lo:hi], kbf[:, lo:hi],
                                (((1,), (1,)), ((), ())),
                                preferred_element_type=jnp.float32)
            sm = jnp.where(ct_pos, s, NEG)
            m_old = m_scr[h]
            m_new = jnp.maximum(m_old, jnp.max(sm, axis=1, keepdims=True))
            alpha = jnp.exp(m_old - m_new)
            p = ct * jnp.exp(sm - m_new)
            l_scr[h] = (l_scr[h] * alpha
                        + jnp.sum(p, axis=1, keepdims=True))
            pv = lax.dot_general(p.astype(jnp.bfloat16), vbf[:, lo:hi],
                                 (((1,), (0,)), ((), ())),
                                 preferred_element_type=jnp.float32)
            acc_scr[h] = acc_scr[h] * alpha + pv
            m_scr[h] = m_new

        @pl.when(pb == NB - 1)
        def _exchange():
            acc_comm[pl.ds(my_z, 1)] = acc_scr[...][None]
            ml_comm[pl.ds(my_z, 1)] = jnp.stack(
                [m_scr[...].reshape(H, B), l_scr[...].reshape(H, B)])[None]

            sends = []
            for dz in range(1, ZDIM):
                zt = (my_z + dz) % ZDIM
                for comm, ssem, rsem in ((acc_comm, acc_send, acc_recv),
                                         (ml_comm, ml_send, ml_recv)):
                    r = pltpu.make_async_remote_copy(
                        src_ref=comm.at[my_z],
                        dst_ref=comm.at[my_z],
                        send_sem=ssem.at[zt],
                        recv_sem=rsem.at[my_z],
                        device_id=(my_x, my_y, zt),
                        device_id_type=pl.DeviceIdType.MESH,
                    )
                    r.start()
                    sends.append(r)

            for dz in range(1, ZDIM):
                zs = (my_z + dz) % ZDIM
                for comm, ssem, rsem in ((acc_comm, acc_send, acc_recv),
                                         (ml_comm, ml_send, ml_recv)):
                    r = pltpu.make_async_remote_copy(
                        src_ref=comm.at[zs],
                        dst_ref=comm.at[zs],
                        send_sem=ssem.at[zs],
                        recv_sem=rsem.at[zs],
                        device_id=(my_x, my_y, zs),
                        device_id_type=pl.DeviceIdType.MESH,
                    )
                    r.wait_recv()
            for r in sends:
                r.wait_send()

            ml = ml_comm[...]
            m_z = ml[:, 0]
            l_z = ml[:, 1]
            mg = jnp.max(m_z, axis=0)
            sc = jnp.exp(m_z - mg[None])
            lg = jnp.sum(l_z * sc, axis=0)
            og = jnp.sum(acc_comm[...] * sc[:, :, :, None], axis=0)
            out = og / lg[:, :, None]
            o_ref[...] = out.transpose(1, 0, 2)[:, None]

    return pl.pallas_call(
        body,
        grid=(NB,),
        in_specs=[
            pl.BlockSpec((B, H * D), lambda i: (0, 0)),
            pl.BlockSpec((BP, PAGE, H * D), lambda i: (i, 0, 0)),
            pl.BlockSpec((BP, PAGE, H * D), lambda i: (i, 0, 0)),
            pl.BlockSpec((B, NSLOTS), lambda i: (0, 0)),
            pl.BlockSpec((B, 1), lambda i: (0, 0)),
        ],
        out_specs=pl.BlockSpec((B, 1, H, D), lambda i: (0, 0, 0, 0)),
        out_shape=jax.ShapeDtypeStruct((B, 1, H, D), jnp.float32),
        scratch_shapes=[
            pltpu.VMEM((H, B, 1), jnp.float32),
            pltpu.VMEM((H, B, 1), jnp.float32),
            pltpu.VMEM((H, B, D), jnp.float32),
            pltpu.VMEM((ZDIM, H, B, D), jnp.float32),
            pltpu.VMEM((ZDIM, 2, H, B), jnp.float32),
            pltpu.SemaphoreType.DMA((ZDIM,)),
            pltpu.SemaphoreType.DMA((ZDIM,)),
            pltpu.SemaphoreType.DMA((ZDIM,)),
            pltpu.SemaphoreType.DMA((ZDIM,)),
        ],
        compiler_params=pltpu.CompilerParams(
            dimension_semantics=("arbitrary",),
        ),
    )(Q2, K2, V2, bt, lens2)


# device time: 43861 ns/iter; 4.2027x vs baseline; 2.7005x over previous
import jax
import jax.numpy as jnp
from jax import lax
from jax.experimental import pallas as pl
from jax.experimental.pallas import tpu as pltpu

B = 8
H = 8
D = 128
PAGE = 16
NPAGES = 512
NSLOTS = 512
ZDIM = 4
BP = 64
NB = NPAGES // BP
NCOL = BP * PAGE * H
BH = B * H
NEG = -1e30
SCALE = D ** -0.5


def kernel(Q, K, V, bt, lens):
    lens2 = lens.reshape(B, 1)

    def body(q_ref, k_ref, v_ref, bt_ref, lens_ref, o_ref,
             m_scr, l_scr, acc_scr, acc_comm, ml_comm,
             acc_send, acc_recv, ml_send, ml_recv):
        pb = pl.program_id(0)
        my_x = lax.axis_index("x")
        my_y = lax.axis_index("y")
        my_z = lax.axis_index("z")

        @pl.when(pb == 0)
        def _init():
            m_scr[...] = jnp.full((BH, 1), NEG, jnp.float32)
            l_scr[...] = jnp.zeros((BH, 1), jnp.float32)
            acc_scr[...] = jnp.zeros((BH, D), jnp.float32)

        bt_ = bt_ref[...]
        lens_ = lens_ref[...]
        base = my_z * NPAGES + pb * BP
        gp = base + lax.broadcasted_iota(jnp.int32, (1, BP, 1), 1)
        slot = lax.broadcasted_iota(jnp.int32, (1, 1, NSLOTS), 2)
        sel = (bt_[:, None, :] == gp) & (slot < lens_[:, :, None])
        count = sel.astype(jnp.int32).sum(axis=2).astype(jnp.float32)
        ct1 = jnp.broadcast_to(
            count[:, :, None], (B, BP, PAGE * H)).reshape(B, NCOL)
        ctb = jnp.broadcast_to(ct1[:, None, :], (B, H, NCOL)).reshape(BH, NCOL)

        ri = lax.broadcasted_iota(jnp.int32, (BH, NCOL), 0)
        ci = lax.broadcasted_iota(jnp.int32, (BH, NCOL), 1)
        diag = (ri % H) == (ci % H)

        qf = (q_ref[...].reshape(BH, D) * SCALE).astype(jnp.bfloat16)
        kf = k_ref[...].reshape(NCOL, D).astype(jnp.bfloat16)
        s_big = lax.dot_general(qf, kf, (((1,), (1,)), ((), ())),
                                preferred_element_type=jnp.float32)
        sm = jnp.where(diag & (ctb > 0.0), s_big, NEG)
        m_old = m_scr[...]
        m_new = jnp.maximum(m_old, jnp.max(sm, axis=1, keepdims=True))
        alpha = jnp.exp(m_old - m_new)
        p = ctb * jnp.exp(sm - m_new)
        l_scr[...] = l_scr[...] * alpha + jnp.sum(p, axis=1, keepdims=True)
        vf = v_ref[...].reshape(NCOL, D).astype(jnp.bfloat16)
        pv = lax.dot_general(p.astype(jnp.bfloat16), vf,
                             (((1,), (0,)), ((), ())),
                             preferred_element_type=jnp.float32)
        acc_scr[...] = acc_scr[...] * alpha + pv
        m_scr[...] = m_new

        @pl.when(pb == NB - 1)
        def _exchange():
            acc_comm[pl.ds(my_z, 1)] = acc_scr[...][None]
            ml_comm[pl.ds(my_z, 1)] = jnp.concatenate(
                [m_scr[...], l_scr[...]], axis=1)[None]

            sends = []
            for dz in range(1, ZDIM):
                zt = (my_z + dz) % ZDIM
                for comm, ssem, rsem in ((acc_comm, acc_send, acc_recv),
                                         (ml_comm, ml_send, ml_recv)):
                    r = pltpu.make_async_remote_copy(
                        src_ref=comm.at[my_z],
                        dst_ref=comm.at[my_z],
                        send_sem=ssem.at[zt],
                        recv_sem=rsem.at[my_z],
                        device_id=(my_x, my_y, zt),
                        device_id_type=pl.DeviceIdType.MESH,
                    )
                    r.start()
                    sends.append(r)

            for dz in range(1, ZDIM):
                zs = (my_z + dz) % ZDIM
                for comm, ssem, rsem in ((acc_comm, acc_send, acc_recv),
                                         (ml_comm, ml_send, ml_recv)):
                    r = pltpu.make_async_remote_copy(
                        src_ref=comm.at[zs],
                        dst_ref=comm.at[zs],
                        send_sem=ssem.at[zs],
                        recv_sem=rsem.at[zs],
                        device_id=(my_x, my_y, zs),
                        device_id_type=pl.DeviceIdType.MESH,
                    )
                    r.wait_recv()
            for r in sends:
                r.wait_send()

            ml = ml_comm[...]
            m_z = ml[:, :, 0:1]
            l_z = ml[:, :, 1:2]
            mg = jnp.max(m_z, axis=0)
            sc = jnp.exp(m_z - mg[None])
            lg = jnp.sum(l_z * sc, axis=0)
            og = jnp.sum(acc_comm[...] * sc, axis=0)
            o_ref[...] = (og / lg).reshape(B, 1, H, D)

    return pl.pallas_call(
        body,
        grid=(NB,),
        in_specs=[
            pl.BlockSpec((B, 1, H, D), lambda i: (0, 0, 0, 0)),
            pl.BlockSpec((BP, PAGE, H, D), lambda i: (i, 0, 0, 0)),
            pl.BlockSpec((BP, PAGE, H, D), lambda i: (i, 0, 0, 0)),
            pl.BlockSpec((B, NSLOTS), lambda i: (0, 0)),
            pl.BlockSpec((B, 1), lambda i: (0, 0)),
        ],
        out_specs=pl.BlockSpec((B, 1, H, D), lambda i: (0, 0, 0, 0)),
        out_shape=jax.ShapeDtypeStruct((B, 1, H, D), jnp.float32),
        scratch_shapes=[
            pltpu.VMEM((BH, 1), jnp.float32),
            pltpu.VMEM((BH, 1), jnp.float32),
            pltpu.VMEM((BH, D), jnp.float32),
            pltpu.VMEM((ZDIM, BH, D), jnp.float32),
            pltpu.VMEM((ZDIM, BH, 2), jnp.float32),
            pltpu.SemaphoreType.DMA((ZDIM,)),
            pltpu.SemaphoreType.DMA((ZDIM,)),
            pltpu.SemaphoreType.DMA((ZDIM,)),
            pltpu.SemaphoreType.DMA((ZDIM,)),
        ],
        compiler_params=pltpu.CompilerParams(
            dimension_semantics=("arbitrary",),
        ),
    )(Q, K, V, bt, lens2)


# device time: 43355 ns/iter; 4.2517x vs baseline; 1.0117x over previous
import jax
import jax.numpy as jnp
from jax import lax
from jax.experimental import pallas as pl
from jax.experimental.pallas import tpu as pltpu

B = 8
H = 8
D = 128
PAGE = 16
NPAGES = 512
NSLOTS = 512
ZDIM = 4
BP = 64
NB = NPAGES // BP
NCOL = BP * PAGE * H
BH = B * H
NEG = -1e30
SCALE = D ** -0.5


def kernel(Q, K, V, bt, lens):
    lens2 = lens.reshape(B, 1)

    def body(q_ref, k_ref, v_ref, bt_ref, lens_ref, o_ref,
             pen_scr, m_scr, l_scr, acc_scr, acc_comm, ml_comm,
             acc_send, acc_recv, ml_send, ml_recv):
        pb = pl.program_id(0)
        my_x = lax.axis_index("x")
        my_y = lax.axis_index("y")
        my_z = lax.axis_index("z")

        @pl.when(pb == 0)
        def _init():
            m_scr[...] = jnp.full((BH, 1), NEG, jnp.float32)
            l_scr[...] = jnp.zeros((BH, 1), jnp.float32)
            acc_scr[...] = jnp.zeros((BH, D), jnp.float32)
            ri = lax.broadcasted_iota(jnp.int32, (BH, NCOL), 0)
            ci = lax.broadcasted_iota(jnp.int32, (BH, NCOL), 1)
            pen_scr[...] = jnp.where((ri % H) == (ci % H), 0.0, NEG)

        bt_ = bt_ref[...]
        lens_ = lens_ref[...]
        base = my_z * NPAGES + pb * BP
        gp = base + lax.broadcasted_iota(jnp.int32, (1, BP, 1), 1)
        slot = lax.broadcasted_iota(jnp.int32, (1, 1, NSLOTS), 2)
        sel = (bt_[:, None, :] == gp) & (slot < lens_[:, :, None])
        count = sel.astype(jnp.int32).sum(axis=2).astype(jnp.float32)
        logct = jnp.where(count > 0.0,
                          jnp.log(jnp.maximum(count, 1.0)), NEG)
        lc1 = jnp.broadcast_to(
            logct[:, :, None], (B, BP, PAGE * H)).reshape(B, NCOL)
        lcb = jnp.broadcast_to(lc1[:, None, :], (B, H, NCOL)).reshape(BH, NCOL)

        qf = (q_ref[...].reshape(BH, D) * SCALE).astype(jnp.bfloat16)
        kf = k_ref[...].reshape(NCOL, D).astype(jnp.bfloat16)
        s_big = lax.dot_general(qf, kf, (((1,), (1,)), ((), ())),
                                preferred_element_type=jnp.float32)
        sm = s_big + lcb + pen_scr[...]
        m_old = m_scr[...]
        m_new = jnp.maximum(m_old, jnp.max(sm, axis=1, keepdims=True))
        alpha = jnp.exp(m_old - m_new)
        p = jnp.exp(sm - m_new)
        l_scr[...] = l_scr[...] * alpha + jnp.sum(p, axis=1, keepdims=True)
        vf = v_ref[...].reshape(NCOL, D).astype(jnp.bfloat16)
        pv = lax.dot_general(p.astype(jnp.bfloat16), vf,
                             (((1,), (0,)), ((), ())),
                             preferred_element_type=jnp.float32)
        acc_scr[...] = acc_scr[...] * alpha + pv
        m_scr[...] = m_new

        @pl.when(pb == NB - 1)
        def _exchange():
            acc_comm[pl.ds(my_z, 1)] = acc_scr[...][None]
            ml_comm[pl.ds(my_z, 1)] = jnp.concatenate(
                [m_scr[...], l_scr[...]], axis=1)[None]

            sends = []
            for dz in range(1, ZDIM):
                zt = (my_z + dz) % ZDIM
                for comm, ssem, rsem in ((acc_comm, acc_send, acc_recv),
                                         (ml_comm, ml_send, ml_recv)):
                    r = pltpu.make_async_remote_copy(
                        src_ref=comm.at[my_z],
                        dst_ref=comm.at[my_z],
                        send_sem=ssem.at[zt],
                        recv_sem=rsem.at[my_z],
                        device_id=(my_x, my_y, zt),
                        device_id_type=pl.DeviceIdType.MESH,
                    )
                    r.start()
                    sends.append(r)

            for dz in range(1, ZDIM):
                zs = (my_z + dz) % ZDIM
                for comm, ssem, rsem in ((acc_comm, acc_send, acc_recv),
                                         (ml_comm, ml_send, ml_recv)):
                    r = pltpu.make_async_remote_copy(
                        src_ref=comm.at[zs],
                        dst_ref=comm.at[zs],
                        send_sem=ssem.at[zs],
                        recv_sem=rsem.at[zs],
                        device_id=(my_x, my_y, zs),
                        device_id_type=pl.DeviceIdType.MESH,
                    )
                    r.wait_recv()
            for r in sends:
                r.wait_send()

            ml = ml_comm[...]
            m_z = ml[:, :, 0:1]
            l_z = ml[:, :, 1:2]
            mg = jnp.max(m_z, axis=0)
            sc = jnp.exp(m_z - mg[None])
            lg = jnp.sum(l_z * sc, axis=0)
            og = jnp.sum(acc_comm[...] * sc, axis=0)
            o_ref[...] = (og / lg).reshape(B, 1, H, D)

    return pl.pallas_call(
        body,
        grid=(NB,),
        in_specs=[
            pl.BlockSpec((B, 1, H, D), lambda i: (0, 0, 0, 0)),
            pl.BlockSpec((BP, PAGE, H, D), lambda i: (i, 0, 0, 0)),
            pl.BlockSpec((BP, PAGE, H, D), lambda i: (i, 0, 0, 0)),
            pl.BlockSpec((B, NSLOTS), lambda i: (0, 0)),
            pl.BlockSpec((B, 1), lambda i: (0, 0)),
        ],
        out_specs=pl.BlockSpec((B, 1, H, D), lambda i: (0, 0, 0, 0)),
        out_shape=jax.ShapeDtypeStruct((B, 1, H, D), jnp.float32),
        scratch_shapes=[
            pltpu.VMEM((BH, NCOL), jnp.float32),
            pltpu.VMEM((BH, 1), jnp.float32),
            pltpu.VMEM((BH, 1), jnp.float32),
            pltpu.VMEM((BH, D), jnp.float32),
            pltpu.VMEM((ZDIM, BH, D), jnp.float32),
            pltpu.VMEM((ZDIM, BH, 2), jnp.float32),
            pltpu.SemaphoreType.DMA((ZDIM,)),
            pltpu.SemaphoreType.DMA((ZDIM,)),
            pltpu.SemaphoreType.DMA((ZDIM,)),
            pltpu.SemaphoreType.DMA((ZDIM,)),
        ],
        compiler_params=pltpu.CompilerParams(
            dimension_semantics=("arbitrary",),
        ),
    )(Q, K, V, bt, lens2)


# device time: 33035 ns/iter; 5.5800x vs baseline; 1.3124x over previous
import jax
import jax.numpy as jnp
from jax import lax
from jax.experimental import pallas as pl
from jax.experimental.pallas import tpu as pltpu

B = 8
H = 8
D = 128
PAGE = 16
NPAGES = 512
NSLOTS = 512
ZDIM = 4
NREP = 8
PPD = NPAGES // NREP
BP = 32
NB = PPD // BP
NCOL = BP * PAGE * H
BH = B * H
NEG = -1e30
SCALE = D ** -0.5

_XY_OFFS = [(dx, dy) for dx in range(2) for dy in range(4) if (dx, dy) != (0, 0)]


def kernel(Q, K, V, bt, lens):
    lens2 = lens.reshape(B, 1)
    my_r = (lax.axis_index("x") * 4 + lax.axis_index("y")).astype(jnp.int32)
    rblk = my_r.reshape(1) * (PPD // BP)

    def body(rblk_ref, q_ref, k_ref, v_ref, bt_ref, lens_ref, o_ref,
             pen_scr, m_scr, l_scr, acc_scr, acc_comm, ml_comm,
             acc_send, acc_recv, ml_send, ml_recv,
             acc_comm2, ml_comm2, acc_send2, acc_recv2, ml_send2, ml_recv2):
        pb = pl.program_id(0)
        my_x = lax.axis_index("x")
        my_y = lax.axis_index("y")
        my_z = lax.axis_index("z")
        my_rep = my_x * 4 + my_y

        @pl.when(pb == 0)
        def _init():
            m_scr[...] = jnp.full((BH, 1), NEG, jnp.float32)
            l_scr[...] = jnp.zeros((BH, 1), jnp.float32)
            acc_scr[...] = jnp.zeros((BH, D), jnp.float32)
            ri = lax.broadcasted_iota(jnp.int32, (BH, NCOL), 0)
            ci = lax.broadcasted_iota(jnp.int32, (BH, NCOL), 1)
            pen_scr[...] = jnp.where((ri % H) == (ci % H), 0.0, NEG)

        bt_ = bt_ref[...]
        lens_ = lens_ref[...]
        base = my_z * NPAGES + my_rep * PPD + pb * BP
        gp = base + lax.broadcasted_iota(jnp.int32, (1, BP, 1), 1)
        slot = lax.broadcasted_iota(jnp.int32, (1, 1, NSLOTS), 2)
        sel = (bt_[:, None, :] == gp) & (slot < lens_[:, :, None])
        count = sel.astype(jnp.int32).sum(axis=2).astype(jnp.float32)
        logct = jnp.where(count > 0.0,
                          jnp.log(jnp.maximum(count, 1.0)), NEG)
        lc1 = jnp.broadcast_to(
            logct[:, :, None], (B, BP, PAGE * H)).reshape(B, NCOL)
        lcb = jnp.broadcast_to(lc1[:, None, :], (B, H, NCOL)).reshape(BH, NCOL)

        qf = (q_ref[...].reshape(BH, D) * SCALE).astype(jnp.bfloat16)
        kf = k_ref[...].reshape(NCOL, D).astype(jnp.bfloat16)
        s_big = lax.dot_general(qf, kf, (((1,), (1,)), ((), ())),
                                preferred_element_type=jnp.float32)
        sm = s_big + lcb + pen_scr[...]
        m_old = m_scr[...]
        m_new = jnp.maximum(m_old, jnp.max(sm, axis=1, keepdims=True))
        alpha = jnp.exp(m_old - m_new)
        p = jnp.exp(sm - m_new)
        l_scr[...] = l_scr[...] * alpha + jnp.sum(p, axis=1, keepdims=True)
        vf = v_ref[...].reshape(NCOL, D).astype(jnp.bfloat16)
        pv = lax.dot_general(p.astype(jnp.bfloat16), vf,
                             (((1,), (0,)), ((), ())),
                             preferred_element_type=jnp.float32)
        acc_scr[...] = acc_scr[...] * alpha + pv
        m_scr[...] = m_new

        @pl.when(pb == NB - 1)
        def _exchange():
            acc_comm[pl.ds(my_z, 1)] = acc_scr[...][None]
            ml_comm[pl.ds(my_z, 1)] = jnp.concatenate(
                [m_scr[...], l_scr[...]], axis=1)[None]

            sends = []
            for dz in range(1, ZDIM):
                zt = (my_z + dz) % ZDIM
                for comm, ssem, rsem in ((acc_comm, acc_send, acc_recv),
                                         (ml_comm, ml_send, ml_recv)):
                    r = pltpu.make_async_remote_copy(
                        src_ref=comm.at[my_z],
                        dst_ref=comm.at[my_z],
                        send_sem=ssem.at[zt],
                        recv_sem=rsem.at[my_z],
                        device_id=(my_x, my_y, zt),
                        device_id_type=pl.DeviceIdType.MESH,
                    )
                    r.start()
                    sends.append(r)

            for dz in range(1, ZDIM):
                zs = (my_z + dz) % ZDIM
                for comm, ssem, rsem in ((acc_comm, acc_send, acc_recv),
                                         (ml_comm, ml_send, ml_recv)):
                    r = pltpu.make_async_remote_copy(
                        src_ref=comm.at[zs],
                        dst_ref=comm.at[zs],
                        send_sem=ssem.at[zs],
                        recv_sem=rsem.at[zs],
                        device_id=(my_x, my_y, zs),
                        device_id_type=pl.DeviceIdType.MESH,
                    )
                    r.wait_recv()
            for r in sends:
                r.wait_send()

            ml = ml_comm[...]
            m_z = ml[:, :, 0:1]
            l_z = ml[:, :, 1:2]
            mg1 = jnp.max(m_z, axis=0)
            sc = jnp.exp(m_z - mg1[None])
            lg1 = jnp.sum(l_z * sc, axis=0)
            og1 = jnp.sum(acc_comm[...] * sc, axis=0)

            acc_comm2[pl.ds(my_rep, 1)] = og1[None]
            ml_comm2[pl.ds(my_rep, 1)] = jnp.concatenate(
                [mg1, lg1], axis=1)[None]

            sends2 = []
            for dx, dy in _XY_OFFS:
                xt = (my_x + dx) % 2
                yt = (my_y + dy) % 4
                rt = xt * 4 + yt
                for comm, ssem, rsem in ((acc_comm2, acc_send2, acc_recv2),
                                         (ml_comm2, ml_send2, ml_recv2)):
                    r = pltpu.make_async_remote_copy(
                        src_ref=comm.at[my_rep],
                        dst_ref=comm.at[my_rep],
                        send_sem=ssem.at[rt],
                        recv_sem=rsem.at[my_rep],
                        device_id=(xt, yt, my_z),
                        device_id_type=pl.DeviceIdType.MESH,
                    )
                    r.start()
                    sends2.append(r)

            for dx, dy in _XY_OFFS:
                xs = (my_x + dx) % 2
                ys = (my_y + dy) % 4
                rs = xs * 4 + ys
                for comm, ssem, rsem in ((acc_comm2, acc_send2, acc_recv2),
                                         (ml_comm2, ml_send2, ml_recv2)):
                    r = pltpu.make_async_remote_copy(
                        src_ref=comm.at[rs],
                        dst_ref=comm.at[rs],
                        send_sem=ssem.at[rs],
                        recv_sem=rsem.at[rs],
                        device_id=(xs, ys, my_z),
                        device_id_type=pl.DeviceIdType.MESH,
                    )
                    r.wait_recv()
            for r in sends2:
                r.wait_send()

            ml2 = ml_comm2[...]
            m_r = ml2[:, :, 0:1]
            l_r = ml2[:, :, 1:2]
            mg = jnp.max(m_r, axis=0)
            sc2 = jnp.exp(m_r - mg[None])
            lg = jnp.sum(l_r * sc2, axis=0)
            og = jnp.sum(acc_comm2[...] * sc2, axis=0)
            o_ref[...] = (og / lg).reshape(B, 1, H, D)

    grid_spec = pltpu.PrefetchScalarGridSpec(
        num_scalar_prefetch=1,
        grid=(NB,),
        in_specs=[
            pl.BlockSpec((B, 1, H, D), lambda i, rb: (0, 0, 0, 0)),
            pl.BlockSpec((BP, PAGE, H, D), lambda i, rb: (rb[0] + i, 0, 0, 0)),
            pl.BlockSpec((BP, PAGE, H, D), lambda i, rb: (rb[0] + i, 0, 0, 0)),
            pl.BlockSpec((B, NSLOTS), lambda i, rb: (0, 0)),
            pl.BlockSpec((B, 1), lambda i, rb: (0, 0)),
        ],
        out_specs=pl.BlockSpec((B, 1, H, D), lambda i, rb: (0, 0, 0, 0)),
        scratch_shapes=[
            pltpu.VMEM((BH, NCOL), jnp.float32),
            pltpu.VMEM((BH, 1), jnp.float32),
            pltpu.VMEM((BH, 1), jnp.float32),
            pltpu.VMEM((BH, D), jnp.float32),
            pltpu.VMEM((ZDIM, BH, D), jnp.float32),
            pltpu.VMEM((ZDIM, BH, 2), jnp.float32),
            pltpu.SemaphoreType.DMA((ZDIM,)),
            pltpu.SemaphoreType.DMA((ZDIM,)),
            pltpu.SemaphoreType.DMA((ZDIM,)),
            pltpu.SemaphoreType.DMA((ZDIM,)),
            pltpu.VMEM((NREP, BH, D), jnp.float32),
            pltpu.VMEM((NREP, BH, 2), jnp.float32),
            pltpu.SemaphoreType.DMA((NREP,)),
            pltpu.SemaphoreType.DMA((NREP,)),
            pltpu.SemaphoreType.DMA((NREP,)),
            pltpu.SemaphoreType.DMA((NREP,)),
        ],
    )

    return pl.pallas_call(
        body,
        grid_spec=grid_spec,
        out_shape=jax.ShapeDtypeStruct((B, 1, H, D), jnp.float32),
        compiler_params=pltpu.CompilerParams(
            dimension_semantics=("arbitrary",),
        ),
    )(rblk, Q, K, V, bt, lens2)
